# baseline (speedup 1.0000x reference)
import functools

import jax
import jax.numpy as jnp
from jax import lax
from jax.experimental import pallas as pl
from jax.experimental.pallas import tpu as pltpu
from jax.experimental.pallas import tpu_sc as plsc

N = 50000
E = 800000
RPT = N // 16
ZB = 125
GB = 128
NGRP = E // GB

_SC_PARAMS = dict(
    mesh=plsc.VectorSubcoreMesh(core_axis_name="c", subcore_axis_name="s"),
    compiler_params=pltpu.CompilerParams(use_tc_tiling_on_sc=False),
)


def _zero_zbuf(zbuf, cols):
    zeros16 = jnp.zeros((16,), jnp.float32)

    def body(r, carry):
        for k in range(cols // 16):
            zbuf[r, pl.ds(k * 16, 16)] = zeros16
        return carry

    lax.fori_loop(0, GB, body, 0)


def _zero_acc(acc, zbuf, sid):
    zsrc = zbuf.at[pl.ds(0, ZB)]
    for k in range(RPT // ZB):
        pltpu.sync_copy(zsrc, acc.at[pl.ds((sid * (RPT // ZB) + k) * ZB, ZB)])


SEC = 32
PADG = 6336


def _pipe_sections(table, srcp, dst2p, acc, stages, gidx, dva, dvb,
                   gsems, ssems, lo, ngrp, npairs):
    def fire_gather(st, b):
        pltpu.async_copy(table.at[gidx.at[pl.ds(b * GB, GB)]],
                         stages[st], gsems[st])

    def wait_gather(st, b):
        pltpu.make_async_copy(table.at[gidx.at[pl.ds(b * GB, GB)]],
                              stages[st], gsems[st]).wait()

    def fire_scatter(st, dv, b):
        pltpu.async_copy(stages[st], acc.at[dv.at[b]], ssems[st], add=True)

    def wait_scatter(st, dv, b):
        pltpu.make_async_copy(stages[st], acc.at[dv.at[b]], ssems[st]).wait()

    def guarded_wait_scatter(k, st, dv, b):
        @pl.when((k >= 0) & (k < ngrp))
        def _():
            wait_scatter(st, dv, b)

    def section(tt, dv, dv_prev):
        kk0 = SEC * tt
        pltpu.sync_copy(srcp.at[pl.ds((lo + kk0) * GB, SEC * GB)], gidx)
        pltpu.sync_copy(dst2p.at[pl.ds(lo + kk0, SEC)], dv)

        guarded_wait_scatter(kk0 - 4, 0, dv_prev, SEC - 4)
        fire_gather(0, 0)
        guarded_wait_scatter(kk0 - 3, 1, dv_prev, SEC - 3)
        fire_gather(1, 1)

        for b in range(SEC):
            st = b % 4
            wait_gather(st, b)

            @pl.when(kk0 + b < ngrp)
            def _(st=st, b=b, dv=dv):
                fire_scatter(st, dv, b)

            if b < SEC - 2:
                st2 = (b + 2) % 4
                if b < 2:
                    guarded_wait_scatter(kk0 + b - 2, st2, dv_prev, b + SEC - 2)
                else:
                    guarded_wait_scatter(kk0 + b - 2, st2, dv, b - 2)
                fire_gather(st2, b + 2)

    def body(j, carry):
        section(2 * j, dva, dvb)
        section(2 * j + 1, dvb, dva)
        return carry

    lax.fori_loop(0, npairs, body, 0)


def _prop_fsplit():
    @functools.partial(
        pl.kernel,
        out_type=[jax.ShapeDtypeStruct((N, 32), jnp.float32)] * 4,
        scratch_types=[
            pltpu.VMEM_SHARED((N, 32), jnp.float32),
            pltpu.VMEM((GB, 32), jnp.float32),
            pltpu.VMEM((GB, 32), jnp.float32),
            pltpu.VMEM((GB, 32), jnp.float32),
            pltpu.VMEM((GB, 32), jnp.float32),
            pltpu.VMEM((SEC * GB,), jnp.int32),
            pltpu.VMEM((SEC, GB), jnp.int32),
            pltpu.VMEM((SEC, GB), jnp.int32),
            pltpu.SemaphoreType.DMA,
            pltpu.SemaphoreType.DMA,
            pltpu.SemaphoreType.DMA,
            pltpu.SemaphoreType.DMA,
            pltpu.SemaphoreType.DMA,
            pltpu.SemaphoreType.DMA,
            pltpu.SemaphoreType.DMA,
            pltpu.SemaphoreType.DMA,
        ],
        **_SC_PARAMS,
    )
    def prop(g0, g1, g2, g3, src_hbm, dst2_hbm, o0, o1, o2, o3,
             acc, st0, st1, st2, st3, gidx, dva, dvb,
             gs0, gs1, gs2, gs3, ss0, ss1, ss2, ss3):
        cid = lax.axis_index("c")
        sid = lax.axis_index("s")
        ngrp = jnp.where(sid < 10, 391, 390)
        lo = sid * 390 + jnp.minimum(sid, 10)
        tables = (g0, g1, g2, g3)
        outs = (o0, o1, o2, o3)

        for p in range(4):
            @pl.when(cid == p // 2)
            def _(p=p):
                _zero_zbuf(st0, 32)
                _zero_acc(acc, st0, sid)
                plsc.subcore_barrier()
                _pipe_sections(tables[p], src_hbm, dst2_hbm, acc,
                               (st0, st1, st2, st3), gidx, dva, dvb,
                               (gs0, gs1, gs2, gs3),
                               (ss0, ss1, ss2, ss3), lo, ngrp, 7)
                plsc.subcore_barrier()
                pltpu.sync_copy(acc.at[pl.ds(sid * RPT, RPT)],
                                outs[p].at[pl.ds(sid * RPT, RPT)])
                plsc.subcore_barrier()

    return prop


def _prop_esplit():
    @functools.partial(
        pl.kernel,
        out_type=[jax.ShapeDtypeStruct((N, 16), jnp.float32)] * 2,
        scratch_types=[
            pltpu.VMEM_SHARED((N, 16), jnp.float32),
            pltpu.VMEM((GB, 16), jnp.float32),
            pltpu.VMEM((GB, 16), jnp.float32),
            pltpu.VMEM((GB, 16), jnp.float32),
            pltpu.VMEM((GB, 16), jnp.float32),
            pltpu.VMEM((SEC * GB,), jnp.int32),
            pltpu.VMEM((SEC, GB), jnp.int32),
            pltpu.VMEM((SEC, GB), jnp.int32),
            pltpu.SemaphoreType.DMA,
            pltpu.SemaphoreType.DMA,
            pltpu.SemaphoreType.DMA,
            pltpu.SemaphoreType.DMA,
            pltpu.SemaphoreType.DMA,
            pltpu.SemaphoreType.DMA,
            pltpu.SemaphoreType.DMA,
            pltpu.SemaphoreType.DMA,
        ],
        **_SC_PARAMS,
    )
    def prop(g_hbm, src_hbm, dst2_hbm, o0, o1,
             acc, st0, st1, st2, st3, gidx, dva, dvb,
             gs0, gs1, gs2, gs3, ss0, ss1, ss2, ss3):
        cid = lax.axis_index("c")
        sid = lax.axis_index("s")
        _zero_zbuf(st0, 16)
        half = NGRP // 2
        ngrp = jnp.where(sid < 5, 196, 195)
        lo = cid * half + sid * 195 + jnp.minimum(sid, 5)

        _zero_acc(acc, st0, sid)
        plsc.subcore_barrier()
        _pipe_sections(g_hbm, src_hbm, dst2_hbm, acc,
                       (st0, st1, st2, st3), gidx, dva, dvb,
                       (gs0, gs1, gs2, gs3),
                       (ss0, ss1, ss2, ss3), lo, ngrp, 4)
        plsc.subcore_barrier()
        for p in range(2):
            @pl.when(cid == p)
            def _(p=p):
                pltpu.sync_copy(acc.at[pl.ds(sid * RPT, RPT)],
                                (o0, o1)[p].at[pl.ds(sid * RPT, RPT)])

    return prop


def _prop_deg():
    @functools.partial(
        pl.kernel,
        out_type=[jax.ShapeDtypeStruct((N, 16), jnp.float32)] * 2,
        scratch_types=[
            pltpu.VMEM_SHARED((N, 16), jnp.float32),
            pltpu.VMEM((GB, 16), jnp.float32),
            pltpu.VMEM((GB, 16), jnp.float32),
            pltpu.VMEM((1, GB), jnp.int32),
        ],
        **_SC_PARAMS,
    )
    def prop(dst2_hbm, o0, o1, acc, stage, zbuf, dstv):
        cid = lax.axis_index("c")
        sid = lax.axis_index("s")
        _zero_zbuf(zbuf, 16)
        ones16 = jnp.ones((16,), jnp.float32)

        def fill(r, carry):
            stage[r, pl.ds(0, 16)] = ones16
            return carry

        lax.fori_loop(0, GB, fill, 0)
        half = NGRP // 2
        ngrp = jnp.where(sid < 5, 196, 195)
        lo = cid * half + sid * 195 + jnp.minimum(sid, 5)

        _zero_acc(acc, zbuf, sid)
        plsc.subcore_barrier()

        def group_body(k, carry):
            pltpu.sync_copy(dst2_hbm.at[pl.ds(lo + k, 1)], dstv)
            pltpu.sync_copy(stage, acc.at[dstv.at[0]], add=True)
            return carry

        lax.fori_loop(0, ngrp, group_body, 0)
        plsc.subcore_barrier()
        for p in range(2):
            @pl.when(cid == p)
            def _(p=p):
                pltpu.sync_copy(acc.at[pl.ds(sid * RPT, RPT)],
                                (o0, o1)[p].at[pl.ds(sid * RPT, RPT)])

    return prop


BM = 400
GRID = N // BM


def _dis(d0b, d1b):
    return lax.rsqrt(d0b[...][:, 0:1] + d1b[...][:, 0:1] + 1.0)


def _spec(cols):
    return pl.BlockSpec((BM, cols), lambda i: (i, 0))


def _split4(res, outs):
    for p, ob in enumerate(outs):
        ob[...] = res[:, 32 * p:32 * p + 32]


def _mm_scale(xp, d0, d1, w):
    kdim = xp.shape[1]

    def body(xb, d0b, d1b, wb, o0, o1, o2, o3):
        res = _dis(d0b, d1b) * jnp.dot(xb[...], wb[...],
                                       preferred_element_type=jnp.float32)
        _split4(res, (o0, o1, o2, o3))

    return pl.pallas_call(
        body,
        grid=(GRID,),
        in_specs=[pl.BlockSpec((BM, kdim), lambda i: (i, 0)),
                  _spec(16), _spec(16),
                  pl.BlockSpec((kdim, 128), lambda i: (0, 0))],
        out_specs=[_spec(32)] * 4,
        out_shape=[jax.ShapeDtypeStruct((N, 32), jnp.float32)] * 4,
    )(xp, d0, d1, w)


def _layer(s4, g4, d0, d1, w, bprev):
    fout = w.shape[1]

    def body(sa, sb, sc, sd, ga, gb_, gc, gd, d0b, d1b, bb, wb, *outs):
        s = jnp.concatenate([sa[...], sb[...], sc[...], sd[...]], axis=1)
        g = jnp.concatenate([ga[...], gb_[...], gc[...], gd[...]], axis=1)
        dis = _dis(d0b, d1b)
        z = jnp.maximum(dis * (s + g) + bb[...], 0.0)
        res = dis * jnp.dot(z, wb[...], preferred_element_type=jnp.float32)
        if fout == 128:
            _split4(res, outs)
        else:
            outs[0][...] = res

    if fout == 128:
        out_specs = [_spec(32)] * 4
        out_shape = [jax.ShapeDtypeStruct((N, 32), jnp.float32)] * 4
    else:
        out_specs = _spec(fout)
        out_shape = jax.ShapeDtypeStruct((N, fout), jnp.float32)

    return pl.pallas_call(
        body,
        grid=(GRID,),
        in_specs=[_spec(32)] * 8 + [
            _spec(16), _spec(16),
            pl.BlockSpec((1, 128), lambda i: (0, 0)),
            pl.BlockSpec((128, fout), lambda i: (0, 0))],
        out_specs=out_specs,
        out_shape=out_shape,
    )(*s4, *g4, d0, d1, bprev, w)


def _final(s0, s1, gp, d0, d1, bcp):
    def body(sa, sb, gb, d0b, d1b, bb, ob):
        logits = _dis(d0b, d1b) * (sa[...] + sb[...] + gb[...]) + bb[...]
        colmask = lax.broadcasted_iota(jnp.int32, (BM, 16), 1) < 10
        masked = jnp.where(colmask, logits, jnp.float32(-1e30))
        m = jnp.max(masked, axis=1, keepdims=True)
        ssum = jnp.sum(jnp.where(colmask, jnp.exp(logits - m), 0.0),
                       axis=1, keepdims=True)
        ob[...] = logits - m - jnp.log(ssum)

    return pl.pallas_call(
        body,
        grid=(GRID,),
        in_specs=[_spec(16)] * 5 + [pl.BlockSpec((1, 16), lambda i: (0, 0))],
        out_specs=_spec(16),
        out_shape=jax.ShapeDtypeStruct((N, 16), jnp.float32),
    )(s0, s1, gp, d0, d1, bcp)


def kernel(x, edge_index, edge_attr, batch, W1, b1, W2, b2, Wc, bc):
    src = jnp.pad(edge_index[0], (0, PADG * GB - E))
    dst2 = jnp.pad(edge_index[1].reshape(NGRP, GB), ((0, PADG - NGRP), (0, 0)))
    wcp = jnp.pad(Wc, ((0, 0), (0, 16 - Wc.shape[1])))
    bcp = jnp.pad(bc, (0, 16 - bc.shape[0])).reshape(1, 16)
    b1r = b1.reshape(1, -1)
    b2r = b2.reshape(1, -1)

    prop_f = _prop_fsplit()
    prop_e = _prop_esplit()
    prop_d = _prop_deg()

    d0, d1 = prop_d(dst2)
    g1 = _mm_scale(x, d0, d1, W1)
    s1 = prop_f(*g1, src, dst2)
    g2 = _layer(s1, g1, d0, d1, W2, b1r)
    s2 = prop_f(*g2, src, dst2)
    g3 = _layer(s2, g2, d0, d1, wcp, b2r)
    s30, s31 = prop_e(g3, src, dst2)
    out16 = _final(s30, s31, g3, d0, d1, bcp)
    return out16[:, :10]


# speedup vs baseline: 1.1995x; 1.1995x over previous
import functools

import jax
import jax.numpy as jnp
from jax import lax
from jax.experimental import pallas as pl
from jax.experimental.pallas import tpu as pltpu
from jax.experimental.pallas import tpu_sc as plsc

N = 50000
E = 800000
RPT = N // 16
ZB = 125
GB = 128
NGRP = E // GB

_SC_PARAMS = dict(
    mesh=plsc.VectorSubcoreMesh(core_axis_name="c", subcore_axis_name="s"),
    compiler_params=pltpu.CompilerParams(use_tc_tiling_on_sc=False),
)


def _zero_zbuf(zbuf, cols):
    zeros16 = jnp.zeros((16,), jnp.float32)

    def body(r, carry):
        for k in range(cols // 16):
            zbuf[r, pl.ds(k * 16, 16)] = zeros16
        return carry

    lax.fori_loop(0, ZB, body, 0)


def _zero_acc(acc, zbuf, sid):
    for k in range(RPT // ZB):
        pltpu.sync_copy(zbuf, acc.at[pl.ds((sid * (RPT // ZB) + k) * ZB, ZB)])


SEC = 16
PADG = 6288


def _pipe_sections(table, srcp, dst2p, acc, stages, gidx, dva, dvb,
                   gsems, ssems, lo, ngrp, npairs):
    def fire_gather(st, b):
        pltpu.async_copy(table.at[gidx.at[pl.ds(b * GB, GB)]],
                         stages[st], gsems[st])

    def wait_gather(st, b):
        pltpu.make_async_copy(table.at[gidx.at[pl.ds(b * GB, GB)]],
                              stages[st], gsems[st]).wait()

    def fire_scatter(st, dv, b):
        pltpu.async_copy(stages[st], acc.at[dv.at[b]], ssems[st], add=True)

    def wait_scatter(st, dv, b):
        pltpu.make_async_copy(stages[st], acc.at[dv.at[b]], ssems[st]).wait()

    def guarded_wait_scatter(k, st, dv, b):
        @pl.when((k >= 0) & (k < ngrp))
        def _():
            wait_scatter(st, dv, b)

    def section(tt, dv, dv_prev):
        kk0 = SEC * tt
        pltpu.sync_copy(srcp.at[pl.ds((lo + kk0) * GB, SEC * GB)], gidx)
        pltpu.sync_copy(dst2p.at[pl.ds(lo + kk0, SEC)], dv)

        guarded_wait_scatter(kk0 - 4, 0, dv_prev, 12)
        fire_gather(0, 0)
        guarded_wait_scatter(kk0 - 3, 1, dv_prev, 13)
        fire_gather(1, 1)

        for b in range(SEC):
            st = b % 4
            wait_gather(st, b)

            @pl.when(kk0 + b < ngrp)
            def _(st=st, b=b, dv=dv):
                fire_scatter(st, dv, b)

            if b < SEC - 2:
                st2 = (b + 2) % 4
                if b < 2:
                    guarded_wait_scatter(kk0 + b - 2, st2, dv_prev, b + 14)
                else:
                    guarded_wait_scatter(kk0 + b - 2, st2, dv, b - 2)
                fire_gather(st2, b + 2)

    def body(j, carry):
        section(2 * j, dva, dvb)
        section(2 * j + 1, dvb, dva)
        return carry

    lax.fori_loop(0, npairs, body, 0)


def _prop_fsplit():
    @functools.partial(
        pl.kernel,
        out_type=[jax.ShapeDtypeStruct((N, 32), jnp.float32)] * 4,
        scratch_types=[
            pltpu.VMEM_SHARED((N, 32), jnp.float32),
            pltpu.VMEM((GB, 32), jnp.float32),
            pltpu.VMEM((GB, 32), jnp.float32),
            pltpu.VMEM((GB, 32), jnp.float32),
            pltpu.VMEM((GB, 32), jnp.float32),
            pltpu.VMEM((ZB, 32), jnp.float32),
            pltpu.VMEM((SEC * GB,), jnp.int32),
            pltpu.VMEM((SEC, GB), jnp.int32),
            pltpu.VMEM((SEC, GB), jnp.int32),
            pltpu.SemaphoreType.DMA,
            pltpu.SemaphoreType.DMA,
            pltpu.SemaphoreType.DMA,
            pltpu.SemaphoreType.DMA,
            pltpu.SemaphoreType.DMA,
            pltpu.SemaphoreType.DMA,
            pltpu.SemaphoreType.DMA,
            pltpu.SemaphoreType.DMA,
        ],
        **_SC_PARAMS,
    )
    def prop(g0, g1, g2, g3, src_hbm, dst2_hbm, o0, o1, o2, o3,
             acc, st0, st1, st2, st3, zbuf, gidx, dva, dvb,
             gs0, gs1, gs2, gs3, ss0, ss1, ss2, ss3):
        cid = lax.axis_index("c")
        sid = lax.axis_index("s")
        _zero_zbuf(zbuf, 32)
        ngrp = jnp.where(sid < 10, 391, 390)
        lo = sid * 390 + jnp.minimum(sid, 10)
        tables = (g0, g1, g2, g3)
        outs = (o0, o1, o2, o3)

        for p in range(4):
            @pl.when(cid == p // 2)
            def _(p=p):
                _zero_acc(acc, zbuf, sid)
                plsc.subcore_barrier()
                _pipe_sections(tables[p], src_hbm, dst2_hbm, acc,
                               (st0, st1, st2, st3), gidx, dva, dvb,
                               (gs0, gs1, gs2, gs3),
                               (ss0, ss1, ss2, ss3), lo, ngrp, 13)
                plsc.subcore_barrier()
                pltpu.sync_copy(acc.at[pl.ds(sid * RPT, RPT)],
                                outs[p].at[pl.ds(sid * RPT, RPT)])
                plsc.subcore_barrier()

    return prop


def _prop_esplit():
    @functools.partial(
        pl.kernel,
        out_type=[jax.ShapeDtypeStruct((N, 16), jnp.float32)] * 2,
        scratch_types=[
            pltpu.VMEM_SHARED((N, 16), jnp.float32),
            pltpu.VMEM((GB, 16), jnp.float32),
            pltpu.VMEM((GB, 16), jnp.float32),
            pltpu.VMEM((GB, 16), jnp.float32),
            pltpu.VMEM((GB, 16), jnp.float32),
            pltpu.VMEM((ZB, 16), jnp.float32),
            pltpu.VMEM((SEC * GB,), jnp.int32),
            pltpu.VMEM((SEC, GB), jnp.int32),
            pltpu.VMEM((SEC, GB), jnp.int32),
            pltpu.SemaphoreType.DMA,
            pltpu.SemaphoreType.DMA,
            pltpu.SemaphoreType.DMA,
            pltpu.SemaphoreType.DMA,
            pltpu.SemaphoreType.DMA,
            pltpu.SemaphoreType.DMA,
            pltpu.SemaphoreType.DMA,
            pltpu.SemaphoreType.DMA,
        ],
        **_SC_PARAMS,
    )
    def prop(g_hbm, src_hbm, dst2_hbm, o0, o1,
             acc, st0, st1, st2, st3, zbuf, gidx, dva, dvb,
             gs0, gs1, gs2, gs3, ss0, ss1, ss2, ss3):
        cid = lax.axis_index("c")
        sid = lax.axis_index("s")
        _zero_zbuf(zbuf, 16)
        half = NGRP // 2
        ngrp = jnp.where(sid < 5, 196, 195)
        lo = cid * half + sid * 195 + jnp.minimum(sid, 5)

        _zero_acc(acc, zbuf, sid)
        plsc.subcore_barrier()
        _pipe_sections(g_hbm, src_hbm, dst2_hbm, acc,
                       (st0, st1, st2, st3), gidx, dva, dvb,
                       (gs0, gs1, gs2, gs3),
                       (ss0, ss1, ss2, ss3), lo, ngrp, 7)
        plsc.subcore_barrier()
        for p in range(2):
            @pl.when(cid == p)
            def _(p=p):
                pltpu.sync_copy(acc.at[pl.ds(sid * RPT, RPT)],
                                (o0, o1)[p].at[pl.ds(sid * RPT, RPT)])

    return prop


def _prop_deg():
    @functools.partial(
        pl.kernel,
        out_type=[jax.ShapeDtypeStruct((N, 16), jnp.float32)] * 2,
        scratch_types=[
            pltpu.VMEM_SHARED((N, 16), jnp.float32),
            pltpu.VMEM((GB, 16), jnp.float32),
            pltpu.VMEM((ZB, 16), jnp.float32),
            pltpu.VMEM((1, GB), jnp.int32),
        ],
        **_SC_PARAMS,
    )
    def prop(dst2_hbm, o0, o1, acc, stage, zbuf, dstv):
        cid = lax.axis_index("c")
        sid = lax.axis_index("s")
        _zero_zbuf(zbuf, 16)
        ones16 = jnp.ones((16,), jnp.float32)

        def fill(r, carry):
            stage[r, pl.ds(0, 16)] = ones16
            return carry

        lax.fori_loop(0, GB, fill, 0)
        half = NGRP // 2
        ngrp = jnp.where(sid < 5, 196, 195)
        lo = cid * half + sid * 195 + jnp.minimum(sid, 5)

        _zero_acc(acc, zbuf, sid)
        plsc.subcore_barrier()

        def group_body(k, carry):
            pltpu.sync_copy(dst2_hbm.at[pl.ds(lo + k, 1)], dstv)
            pltpu.sync_copy(stage, acc.at[dstv.at[0]], add=True)
            return carry

        lax.fori_loop(0, ngrp, group_body, 0)
        plsc.subcore_barrier()
        for p in range(2):
            @pl.when(cid == p)
            def _(p=p):
                pltpu.sync_copy(acc.at[pl.ds(sid * RPT, RPT)],
                                (o0, o1)[p].at[pl.ds(sid * RPT, RPT)])

    return prop


BM = 1000
GRID = N // BM


def _dis(d0b, d1b):
    return lax.rsqrt(d0b[...][:, 0:1] + d1b[...][:, 0:1] + 1.0)


def _spec(cols):
    return pl.BlockSpec((BM, cols), lambda i: (i, 0))


def _split4(res, outs):
    for p, ob in enumerate(outs):
        ob[...] = res[:, 32 * p:32 * p + 32]


def _mm_scale(xp, d0, d1, w):
    kdim = xp.shape[1]

    def body(xb, d0b, d1b, wb, o0, o1, o2, o3):
        res = _dis(d0b, d1b) * jnp.dot(xb[...], wb[...],
                                       preferred_element_type=jnp.float32)
        _split4(res, (o0, o1, o2, o3))

    return pl.pallas_call(
        body,
        grid=(GRID,),
        in_specs=[pl.BlockSpec((BM, kdim), lambda i: (i, 0)),
                  _spec(16), _spec(16),
                  pl.BlockSpec((kdim, 128), lambda i: (0, 0))],
        out_specs=[_spec(32)] * 4,
        out_shape=[jax.ShapeDtypeStruct((N, 32), jnp.float32)] * 4,
    )(xp, d0, d1, w)


def _layer(s4, g4, d0, d1, w, bprev):
    fout = w.shape[1]

    def body(sa, sb, sc, sd, ga, gb_, gc, gd, d0b, d1b, bb, wb, *outs):
        s = jnp.concatenate([sa[...], sb[...], sc[...], sd[...]], axis=1)
        g = jnp.concatenate([ga[...], gb_[...], gc[...], gd[...]], axis=1)
        dis = _dis(d0b, d1b)
        z = jnp.maximum(dis * (s + g) + bb[...], 0.0)
        res = dis * jnp.dot(z, wb[...], preferred_element_type=jnp.float32)
        if fout == 128:
            _split4(res, outs)
        else:
            outs[0][...] = res

    if fout == 128:
        out_specs = [_spec(32)] * 4
        out_shape = [jax.ShapeDtypeStruct((N, 32), jnp.float32)] * 4
    else:
        out_specs = _spec(fout)
        out_shape = jax.ShapeDtypeStruct((N, fout), jnp.float32)

    return pl.pallas_call(
        body,
        grid=(GRID,),
        in_specs=[_spec(32)] * 8 + [
            _spec(16), _spec(16),
            pl.BlockSpec((1, 128), lambda i: (0, 0)),
            pl.BlockSpec((128, fout), lambda i: (0, 0))],
        out_specs=out_specs,
        out_shape=out_shape,
    )(*s4, *g4, d0, d1, bprev, w)


def _final(s0, s1, gp, d0, d1, bcp):
    def body(sa, sb, gb, d0b, d1b, bb, ob):
        logits = _dis(d0b, d1b) * (sa[...] + sb[...] + gb[...]) + bb[...]
        colmask = lax.broadcasted_iota(jnp.int32, (BM, 16), 1) < 10
        masked = jnp.where(colmask, logits, jnp.float32(-1e30))
        m = jnp.max(masked, axis=1, keepdims=True)
        ssum = jnp.sum(jnp.where(colmask, jnp.exp(logits - m), 0.0),
                       axis=1, keepdims=True)
        ob[...] = logits - m - jnp.log(ssum)

    return pl.pallas_call(
        body,
        grid=(GRID,),
        in_specs=[_spec(16)] * 5 + [pl.BlockSpec((1, 16), lambda i: (0, 0))],
        out_specs=_spec(16),
        out_shape=jax.ShapeDtypeStruct((N, 16), jnp.float32),
    )(s0, s1, gp, d0, d1, bcp)


def kernel(x, edge_index, edge_attr, batch, W1, b1, W2, b2, Wc, bc):
    src = jnp.pad(edge_index[0], (0, PADG * GB - E))
    dst2 = jnp.pad(edge_index[1].reshape(NGRP, GB), ((0, PADG - NGRP), (0, 0)))
    wcp = jnp.pad(Wc, ((0, 0), (0, 16 - Wc.shape[1])))
    bcp = jnp.pad(bc, (0, 16 - bc.shape[0])).reshape(1, 16)
    b1r = b1.reshape(1, -1)
    b2r = b2.reshape(1, -1)

    prop_f = _prop_fsplit()
    prop_e = _prop_esplit()
    prop_d = _prop_deg()

    d0, d1 = prop_d(dst2)
    g1 = _mm_scale(x, d0, d1, W1)
    s1 = prop_f(*g1, src, dst2)
    g2 = _layer(s1, g1, d0, d1, W2, b1r)
    s2 = prop_f(*g2, src, dst2)
    g3 = _layer(s2, g2, d0, d1, wcp, b2r)
    s30, s31 = prop_e(g3, src, dst2)
    out16 = _final(s30, s31, g3, d0, d1, bcp)
    return out16[:, :10]
